# Initial kernel scaffold; baseline (speedup 1.0000x reference)
import jax
import jax.numpy as jnp
from jax.experimental import pallas as pl


def kernel(x, edge_index, W1, b1, W2, b2):
    raise NotImplementedError("write your pallas kernel here")


# speedup vs baseline: 74.0071x; 74.0071x over previous
import functools

import jax
import jax.numpy as jnp
from jax import lax
from jax.experimental import pallas as pl
from jax.experimental.pallas import tpu as pltpu
from jax.experimental.pallas import tpu_sc as plsc

NC = 2
NS = 16
LW = 128


def _sc_mesh():
    return plsc.VectorSubcoreMesh(
        core_axis_name="c", subcore_axis_name="s", num_cores=NC, num_subcores=NS
    )


def _make_deg_kernel(n_pad, rows_w, kb):
    npt = n_pad // NS
    n_iter = rows_w // kb

    @functools.partial(
        pl.kernel,
        out_type=jax.ShapeDtypeStruct((NC, n_pad), jnp.float32),
        mesh=_sc_mesh(),
        scratch_types=[
            pltpu.VMEM((kb, LW), jnp.int32),
            pltpu.VMEM((kb, LW), jnp.float32),
            pltpu.VMEM_SHARED((n_pad,), jnp.float32),
        ],
    )
    def deg_kernel(dst_hbm, ones_hbm, zeros_hbm, out_hbm, didx, ones_v, t_sh):
        c = lax.axis_index("c")
        s = lax.axis_index("s")
        w = c * NS + s
        pltpu.sync_copy(zeros_hbm.at[pl.ds(s * npt, npt)], t_sh.at[pl.ds(s * npt, npt)])
        pltpu.sync_copy(ones_hbm, ones_v)
        plsc.subcore_barrier()
        rbase = w * rows_w

        def body(i, carry):
            r = rbase + i * kb
            pltpu.sync_copy(dst_hbm.at[pl.ds(r, kb)], didx)
            for j in range(kb):
                pltpu.sync_copy(ones_v.at[j], t_sh.at[didx.at[j]], add=True)
            return carry

        lax.fori_loop(0, n_iter, body, 0)
        plsc.subcore_barrier()
        pltpu.sync_copy(t_sh.at[pl.ds(s * npt, npt)], out_hbm.at[c, pl.ds(s * npt, npt)])

    return deg_kernel


def _make_edge_kernel(n_pad, rows_w, kb, nch):
    npt = n_pad // NS
    n_iter = rows_w // kb
    out_t = tuple(jax.ShapeDtypeStruct((NC, n_pad), jnp.float32) for _ in range(nch))
    scratch = (
        [pltpu.VMEM((kb, LW), jnp.int32) for _ in range(2)]
        + [pltpu.VMEM((kb, LW), jnp.float32) for _ in range(nch)]
        + [pltpu.VMEM_SHARED((n_pad,), jnp.float32) for _ in range(2 * nch)]
    )

    @functools.partial(
        pl.kernel,
        out_type=out_t[0] if nch == 1 else out_t,
        mesh=_sc_mesh(),
        scratch_types=scratch,
    )
    def edge_kernel(src_hbm, dst_hbm, *rest):
        g_hbm = rest[:nch]
        zeros_hbm = rest[nch]
        out_hbm = rest[nch + 1 : nch + 1 + nch]
        sidx, didx = rest[nch + 1 + nch : nch + 3 + nch]
        vals = rest[nch + 3 + nch : nch + 3 + 2 * nch]
        g_sh = rest[nch + 3 + 2 * nch : nch + 3 + 3 * nch]
        t_sh = rest[nch + 3 + 3 * nch :]

        c = lax.axis_index("c")
        s = lax.axis_index("s")
        w = c * NS + s
        sl = pl.ds(s * npt, npt)
        for ch in range(nch):
            pltpu.sync_copy(g_hbm[ch].at[sl], g_sh[ch].at[sl])
            pltpu.sync_copy(zeros_hbm.at[sl], t_sh[ch].at[sl])
        plsc.subcore_barrier()
        rbase = w * rows_w

        def body(i, carry):
            r = rbase + i * kb
            pltpu.sync_copy(src_hbm.at[pl.ds(r, kb)], sidx)
            pltpu.sync_copy(dst_hbm.at[pl.ds(r, kb)], didx)
            for ch in range(nch):
                for j in range(kb):
                    pltpu.sync_copy(g_sh[ch].at[sidx.at[j]], vals[ch].at[j])
                for j in range(kb):
                    pltpu.sync_copy(vals[ch].at[j], t_sh[ch].at[didx.at[j]], add=True)
            return carry

        lax.fori_loop(0, n_iter, body, 0)
        plsc.subcore_barrier()
        for ch in range(nch):
            pltpu.sync_copy(t_sh[ch].at[sl], out_hbm[ch].at[c, sl])

    return edge_kernel


def _tc_norm(degp, xp):
    def body(degp_ref, x_ref, dinv_ref, g_ref):
        deg = degp_ref[0] + degp_ref[1] + 1.0
        dinv = lax.rsqrt(deg)
        dinv_ref[...] = dinv
        g_ref[...] = dinv * x_ref[...]

    r = xp.shape[0]
    return pl.pallas_call(
        body,
        out_shape=(
            jax.ShapeDtypeStruct((r, 128), jnp.float32),
            jax.ShapeDtypeStruct((r, 128), jnp.float32),
        ),
    )(degp, xp)


def _tc_layer1(tp, g, dinv, W1, b1, W2):
    def body(tp_ref, g_ref, dinv_ref, w1_ref, b1_ref, w2_ref, a_ref, b_ref):
        dinv = dinv_ref[...]
        s1 = dinv * (tp_ref[0] + tp_ref[1] + g_ref[...])
        acc_a = jnp.zeros_like(s1)
        acc_b = jnp.zeros_like(s1)
        for k in range(16):
            h = jnp.maximum(s1 * w1_ref[0, k] + b1_ref[0, k], 0.0)
            acc_a = acc_a + h * w2_ref[k, 0]
            acc_b = acc_b + h * w2_ref[k, 1]
        a_ref[...] = dinv * acc_a
        b_ref[...] = dinv * acc_b

    r = g.shape[0]
    smem = pl.BlockSpec(memory_space=pltpu.MemorySpace.SMEM)
    vmem = pl.BlockSpec(memory_space=pltpu.MemorySpace.VMEM)
    return pl.pallas_call(
        body,
        in_specs=[vmem, vmem, vmem, smem, smem, smem],
        out_shape=(
            jax.ShapeDtypeStruct((r, 128), jnp.float32),
            jax.ShapeDtypeStruct((r, 128), jnp.float32),
        ),
    )(tp, g, dinv, W1, b1, W2)


def _tc_final(t2ap, t2bp, g2a, g2b, dinv, b2):
    def body(ta_ref, tb_ref, ga_ref, gb_ref, dinv_ref, b2_ref, oa_ref, ob_ref):
        dinv = dinv_ref[...]
        a = dinv * (ta_ref[0] + ta_ref[1] + ga_ref[...]) + b2_ref[0, 0]
        b = dinv * (tb_ref[0] + tb_ref[1] + gb_ref[...]) + b2_ref[0, 1]
        m = jnp.maximum(a, b)
        lse = m + jnp.log(jnp.exp(a - m) + jnp.exp(b - m))
        oa_ref[...] = a - lse
        ob_ref[...] = b - lse

    r = g2a.shape[0]
    smem = pl.BlockSpec(memory_space=pltpu.MemorySpace.SMEM)
    vmem = pl.BlockSpec(memory_space=pltpu.MemorySpace.VMEM)
    return pl.pallas_call(
        body,
        in_specs=[vmem, vmem, vmem, vmem, vmem, smem],
        out_shape=(
            jax.ShapeDtypeStruct((r, 128), jnp.float32),
            jax.ShapeDtypeStruct((r, 128), jnp.float32),
        ),
    )(t2ap, t2bp, g2a, g2b, dinv, b2)


def kernel(x, edge_index, W1, b1, W2, b2):
    n = x.shape[0]
    e = edge_index.shape[1]
    n_pad = ((n + 1023) // 1024) * 1024
    r = n_pad // 128
    nw = NC * NS
    rows_w = ((e + nw * LW - 1) // (nw * LW) + 7) // 8 * 8
    e_pad = nw * rows_w * LW

    src = edge_index[0].astype(jnp.int32)
    dst = edge_index[1].astype(jnp.int32)
    pad = e_pad - e
    pad_idx = (n + jnp.arange(pad, dtype=jnp.int32) % (n_pad - n)).astype(jnp.int32)
    src2 = jnp.concatenate([src, pad_idx]).reshape(-1, LW)
    dst2 = jnp.concatenate([dst, pad_idx]).reshape(-1, LW)

    zeros_np = jnp.zeros((n_pad,), jnp.float32)
    xp = jnp.pad(x[:, 0], (0, n_pad - n)).reshape(r, 128)

    kb1 = 8
    ones_rows = jnp.ones((kb1, LW), jnp.float32)
    degp = _make_deg_kernel(n_pad, rows_w, kb1)(dst2, ones_rows, zeros_np)

    dinv, g = _tc_norm(degp.reshape(NC, r, 128), xp)

    tp = _make_edge_kernel(n_pad, rows_w, 8, 1)(src2, dst2, g.reshape(n_pad), zeros_np)

    g2a, g2b = _tc_layer1(
        tp.reshape(NC, r, 128), g, dinv,
        W1.reshape(1, 16), b1.reshape(1, 16), W2,
    )

    t2ap, t2bp = _make_edge_kernel(n_pad, rows_w, 4, 2)(
        src2, dst2, g2a.reshape(n_pad), g2b.reshape(n_pad), zeros_np
    )

    oa, ob = _tc_final(
        t2ap.reshape(NC, r, 128), t2bp.reshape(NC, r, 128),
        g2a, g2b, dinv, b2.reshape(1, 2),
    )
    return jnp.stack([oa.reshape(n_pad)[:n], ob.reshape(n_pad)[:n]], axis=1)


# speedup vs baseline: 152.2654x; 2.0574x over previous
import functools

import jax
import jax.numpy as jnp
from jax import lax
from jax.experimental import pallas as pl
from jax.experimental.pallas import tpu as pltpu
from jax.experimental.pallas import tpu_sc as plsc

NC = 2
NS = 16


def _sc_mesh():
    return plsc.VectorSubcoreMesh(
        core_axis_name="c", subcore_axis_name="s", num_cores=NC, num_subcores=NS
    )


def _make_deg_kernel(n_pad, epw, c_sz):
    npt = n_pad // NS
    n_iter = epw // c_sz

    @functools.partial(
        pl.kernel,
        out_type=jax.ShapeDtypeStruct((NC, n_pad), jnp.float32),
        mesh=_sc_mesh(),
        scratch_types=[
            pltpu.VMEM((c_sz,), jnp.int32),
            pltpu.VMEM((c_sz,), jnp.float32),
            pltpu.VMEM_SHARED((n_pad,), jnp.float32),
        ],
    )
    def deg_kernel(dst_hbm, ones_hbm, zeros_hbm, out_hbm, didx, ones_v, t_sh):
        c = lax.axis_index("c")
        s = lax.axis_index("s")
        w = c * NS + s
        pltpu.sync_copy(zeros_hbm.at[pl.ds(s * npt, npt)], t_sh.at[pl.ds(s * npt, npt)])
        pltpu.sync_copy(ones_hbm, ones_v)
        plsc.subcore_barrier()
        base = w * epw

        def body(i, carry):
            pltpu.sync_copy(dst_hbm.at[pl.ds(base + i * c_sz, c_sz)], didx)
            pltpu.sync_copy(ones_v, t_sh.at[didx], add=True)
            return carry

        lax.fori_loop(0, n_iter, body, 0)
        plsc.subcore_barrier()
        pltpu.sync_copy(t_sh.at[pl.ds(s * npt, npt)], out_hbm.at[c, pl.ds(s * npt, npt)])

    return deg_kernel


def _make_edge_kernel(n_pad, epw, c_sz, nch):
    npt = n_pad // NS
    n_iter = epw // c_sz
    out_t = tuple(jax.ShapeDtypeStruct((NC, n_pad), jnp.float32) for _ in range(nch))
    scratch = (
        [pltpu.VMEM((c_sz,), jnp.int32) for _ in range(2)]
        + [pltpu.VMEM((c_sz,), jnp.float32) for _ in range(nch)]
        + [pltpu.VMEM_SHARED((n_pad,), jnp.float32) for _ in range(2 * nch)]
    )

    @functools.partial(
        pl.kernel,
        out_type=out_t[0] if nch == 1 else out_t,
        mesh=_sc_mesh(),
        scratch_types=scratch,
    )
    def edge_kernel(src_hbm, dst_hbm, *rest):
        g_hbm = rest[:nch]
        zeros_hbm = rest[nch]
        out_hbm = rest[nch + 1 : nch + 1 + nch]
        sidx, didx = rest[nch + 1 + nch : nch + 3 + nch]
        vals = rest[nch + 3 + nch : nch + 3 + 2 * nch]
        g_sh = rest[nch + 3 + 2 * nch : nch + 3 + 3 * nch]
        t_sh = rest[nch + 3 + 3 * nch :]

        c = lax.axis_index("c")
        s = lax.axis_index("s")
        w = c * NS + s
        sl = pl.ds(s * npt, npt)
        for ch in range(nch):
            pltpu.sync_copy(g_hbm[ch].at[sl], g_sh[ch].at[sl])
            pltpu.sync_copy(zeros_hbm.at[sl], t_sh[ch].at[sl])
        plsc.subcore_barrier()
        base = w * epw

        def body(i, carry):
            o = base + i * c_sz
            pltpu.sync_copy(src_hbm.at[pl.ds(o, c_sz)], sidx)
            pltpu.sync_copy(dst_hbm.at[pl.ds(o, c_sz)], didx)
            for ch in range(nch):
                pltpu.sync_copy(g_sh[ch].at[sidx], vals[ch])
                pltpu.sync_copy(vals[ch], t_sh[ch].at[didx], add=True)
            return carry

        lax.fori_loop(0, n_iter, body, 0)
        plsc.subcore_barrier()
        for ch in range(nch):
            pltpu.sync_copy(t_sh[ch].at[sl], out_hbm[ch].at[c, sl])

    return edge_kernel


def _tc_norm(degp, xp):
    def body(degp_ref, x_ref, dinv_ref, g_ref):
        deg = degp_ref[0] + degp_ref[1] + 1.0
        dinv = lax.rsqrt(deg)
        dinv_ref[...] = dinv
        g_ref[...] = dinv * x_ref[...]

    r = xp.shape[0]
    return pl.pallas_call(
        body,
        out_shape=(
            jax.ShapeDtypeStruct((r, 128), jnp.float32),
            jax.ShapeDtypeStruct((r, 128), jnp.float32),
        ),
    )(degp, xp)


def _tc_layer1(tp, g, dinv, W1, b1, W2):
    def body(tp_ref, g_ref, dinv_ref, w1_ref, b1_ref, w2_ref, a_ref, b_ref):
        dinv = dinv_ref[...]
        s1 = dinv * (tp_ref[0] + tp_ref[1] + g_ref[...])
        acc_a = jnp.zeros_like(s1)
        acc_b = jnp.zeros_like(s1)
        for k in range(16):
            h = jnp.maximum(s1 * w1_ref[0, k] + b1_ref[0, k], 0.0)
            acc_a = acc_a + h * w2_ref[k, 0]
            acc_b = acc_b + h * w2_ref[k, 1]
        a_ref[...] = dinv * acc_a
        b_ref[...] = dinv * acc_b

    r = g.shape[0]
    smem = pl.BlockSpec(memory_space=pltpu.MemorySpace.SMEM)
    vmem = pl.BlockSpec(memory_space=pltpu.MemorySpace.VMEM)
    return pl.pallas_call(
        body,
        in_specs=[vmem, vmem, vmem, smem, smem, smem],
        out_shape=(
            jax.ShapeDtypeStruct((r, 128), jnp.float32),
            jax.ShapeDtypeStruct((r, 128), jnp.float32),
        ),
    )(tp, g, dinv, W1, b1, W2)


def _tc_final(t2ap, t2bp, g2a, g2b, dinv, b2):
    def body(ta_ref, tb_ref, ga_ref, gb_ref, dinv_ref, b2_ref, oa_ref, ob_ref):
        dinv = dinv_ref[...]
        a = dinv * (ta_ref[0] + ta_ref[1] + ga_ref[...]) + b2_ref[0, 0]
        b = dinv * (tb_ref[0] + tb_ref[1] + gb_ref[...]) + b2_ref[0, 1]
        m = jnp.maximum(a, b)
        lse = m + jnp.log(jnp.exp(a - m) + jnp.exp(b - m))
        oa_ref[...] = a - lse
        ob_ref[...] = b - lse

    r = g2a.shape[0]
    smem = pl.BlockSpec(memory_space=pltpu.MemorySpace.SMEM)
    vmem = pl.BlockSpec(memory_space=pltpu.MemorySpace.VMEM)
    return pl.pallas_call(
        body,
        in_specs=[vmem, vmem, vmem, vmem, vmem, smem],
        out_shape=(
            jax.ShapeDtypeStruct((r, 128), jnp.float32),
            jax.ShapeDtypeStruct((r, 128), jnp.float32),
        ),
    )(t2ap, t2bp, g2a, g2b, dinv, b2)


def kernel(x, edge_index, W1, b1, W2, b2):
    n = x.shape[0]
    e = edge_index.shape[1]
    n_pad = ((n + 1023) // 1024) * 1024
    r = n_pad // 128
    nw = NC * NS
    c_sz = 2048
    epw = ((e + nw - 1) // nw + c_sz - 1) // c_sz * c_sz
    e_pad = nw * epw

    src = edge_index[0].astype(jnp.int32)
    dst = edge_index[1].astype(jnp.int32)
    pad = e_pad - e
    pad_idx = (n + jnp.arange(pad, dtype=jnp.int32) % (n_pad - n)).astype(jnp.int32)
    src_p = jnp.concatenate([src, pad_idx])
    dst_p = jnp.concatenate([dst, pad_idx])

    zeros_np = jnp.zeros((n_pad,), jnp.float32)
    xp = jnp.pad(x[:, 0], (0, n_pad - n)).reshape(r, 128)

    ones_c = jnp.ones((c_sz,), jnp.float32)
    degp = _make_deg_kernel(n_pad, epw, c_sz)(dst_p, ones_c, zeros_np)

    dinv, g = _tc_norm(degp.reshape(NC, r, 128), xp)

    tp = _make_edge_kernel(n_pad, epw, c_sz, 1)(src_p, dst_p, g.reshape(n_pad), zeros_np)

    g2a, g2b = _tc_layer1(
        tp.reshape(NC, r, 128), g, dinv,
        W1.reshape(1, 16), b1.reshape(1, 16), W2,
    )

    t2ap, t2bp = _make_edge_kernel(n_pad, epw, c_sz, 2)(
        src_p, dst_p, g2a.reshape(n_pad), g2b.reshape(n_pad), zeros_np
    )

    oa, ob = _tc_final(
        t2ap.reshape(NC, r, 128), t2bp.reshape(NC, r, 128),
        g2a, g2b, dinv, b2.reshape(1, 2),
    )
    return jnp.stack([oa.reshape(n_pad)[:n], ob.reshape(n_pad)[:n]], axis=1)


# speedup vs baseline: 250.9915x; 1.6484x over previous
import functools

import jax
import jax.numpy as jnp
from jax import lax
from jax.experimental import pallas as pl
from jax.experimental.pallas import tpu as pltpu
from jax.experimental.pallas import tpu_sc as plsc

NC = 2
NS = 16


def _sc_mesh():
    return plsc.VectorSubcoreMesh(
        core_axis_name="c", subcore_axis_name="s", num_cores=NC, num_subcores=NS
    )


def _make_deg_kernel(n_pad, epw, c_sz, nb):
    npt = n_pad // NS
    n_grp = epw // (c_sz * nb)

    @functools.partial(
        pl.kernel,
        out_type=jax.ShapeDtypeStruct((NC, n_pad), jnp.float32),
        mesh=_sc_mesh(),
        scratch_types=(
            [pltpu.VMEM((c_sz,), jnp.int32) for _ in range(nb)]
            + [pltpu.VMEM((c_sz,), jnp.float32)]
            + [pltpu.VMEM_SHARED((n_pad,), jnp.float32)]
            + [pltpu.SemaphoreType.DMA for _ in range(2 * nb)]
        ),
    )
    def deg_kernel(dst_hbm, ones_hbm, zeros_hbm, out_hbm, *rest):
        didx = rest[:nb]
        ones_v = rest[nb]
        t_sh = rest[nb + 1]
        lsem = rest[nb + 2 : nb + 2 + nb]
        ssem = rest[nb + 2 + nb :]

        c = lax.axis_index("c")
        s = lax.axis_index("s")
        w = c * NS + s
        pltpu.sync_copy(zeros_hbm.at[pl.ds(s * npt, npt)], t_sh.at[pl.ds(s * npt, npt)])
        pltpu.sync_copy(ones_hbm, ones_v)
        plsc.subcore_barrier()
        base = w * epw

        def body(i, carry):
            o = base + i * (c_sz * nb)
            ld = [
                pltpu.async_copy(
                    dst_hbm.at[pl.ds(o + j * c_sz, c_sz)], didx[j], lsem[j]
                )
                for j in range(nb)
            ]
            st = []
            for j in range(nb):
                ld[j].wait()
                st.append(
                    pltpu.async_copy(ones_v, t_sh.at[didx[j]], ssem[j], add=True)
                )
            for d in st:
                d.wait()
            return carry

        lax.fori_loop(0, n_grp, body, 0)
        plsc.subcore_barrier()
        pltpu.sync_copy(t_sh.at[pl.ds(s * npt, npt)], out_hbm.at[c, pl.ds(s * npt, npt)])

    return deg_kernel


def _make_edge_kernel(n_pad, epw, c_sz, nch, nb):
    npt = n_pad // NS
    n_grp = epw // (c_sz * nb)
    out_t = tuple(jax.ShapeDtypeStruct((NC, n_pad), jnp.float32) for _ in range(nch))
    scratch = (
        [pltpu.VMEM((c_sz,), jnp.int32) for _ in range(2 * nb)]
        + [pltpu.VMEM((c_sz,), jnp.float32) for _ in range(nch * nb)]
        + [pltpu.VMEM_SHARED((n_pad,), jnp.float32) for _ in range(2 * nch)]
        + [pltpu.SemaphoreType.DMA for _ in range(4 * nb)]
    )

    @functools.partial(
        pl.kernel,
        out_type=out_t[0] if nch == 1 else out_t,
        mesh=_sc_mesh(),
        scratch_types=scratch,
    )
    def edge_kernel(src_hbm, dst_hbm, *rest):
        g_hbm = rest[:nch]
        zeros_hbm = rest[nch]
        out_hbm = rest[nch + 1 : nch + 1 + nch]
        sc = rest[nch + 1 + nch :]
        sidx = sc[:nb]
        didx = sc[nb : 2 * nb]
        vals = sc[2 * nb : 2 * nb + nch * nb]
        g_sh = sc[2 * nb + nch * nb : 2 * nb + nch * nb + nch]
        t_sh = sc[2 * nb + nch * nb + nch : 2 * nb + nch * nb + 2 * nch]
        sems = sc[2 * nb + nch * nb + 2 * nch :]
        lsem = sems[: 2 * nb]
        gsem = sems[2 * nb : 3 * nb]
        ssem = sems[3 * nb :]

        c = lax.axis_index("c")
        s = lax.axis_index("s")
        w = c * NS + s
        sl = pl.ds(s * npt, npt)
        for ch in range(nch):
            pltpu.sync_copy(g_hbm[ch].at[sl], g_sh[ch].at[sl])
            pltpu.sync_copy(zeros_hbm.at[sl], t_sh[ch].at[sl])
        plsc.subcore_barrier()
        base = w * epw

        def body(i, carry):
            o = base + i * (c_sz * nb)
            lds = []
            for j in range(nb):
                oj = o + j * c_sz
                lds.append((
                    pltpu.async_copy(src_hbm.at[pl.ds(oj, c_sz)], sidx[j], lsem[2 * j]),
                    pltpu.async_copy(dst_hbm.at[pl.ds(oj, c_sz)], didx[j], lsem[2 * j + 1]),
                ))
            gts = []
            for j in range(nb):
                lds[j][0].wait()
                gts.append([
                    pltpu.async_copy(g_sh[ch].at[sidx[j]], vals[ch * nb + j], gsem[j])
                    for ch in range(nch)
                ])
            sts = []
            for j in range(nb):
                for d in gts[j]:
                    d.wait()
                lds[j][1].wait()
                sts.append([
                    pltpu.async_copy(
                        vals[ch * nb + j], t_sh[ch].at[didx[j]], ssem[j], add=True
                    )
                    for ch in range(nch)
                ])
            for ds in sts:
                for d in ds:
                    d.wait()
            return carry

        lax.fori_loop(0, n_grp, body, 0)
        plsc.subcore_barrier()
        for ch in range(nch):
            pltpu.sync_copy(t_sh[ch].at[sl], out_hbm[ch].at[c, sl])

    return edge_kernel


def _tc_norm(degp, xp):
    def body(degp_ref, x_ref, dinv_ref, g_ref):
        deg = degp_ref[0] + degp_ref[1] + 1.0
        dinv = lax.rsqrt(deg)
        dinv_ref[...] = dinv
        g_ref[...] = dinv * x_ref[...]

    r = xp.shape[0]
    return pl.pallas_call(
        body,
        out_shape=(
            jax.ShapeDtypeStruct((r, 128), jnp.float32),
            jax.ShapeDtypeStruct((r, 128), jnp.float32),
        ),
    )(degp, xp)


def _tc_layer1(tp, g, dinv, W1, b1, W2):
    def body(tp_ref, g_ref, dinv_ref, w1_ref, b1_ref, w2_ref, a_ref, b_ref):
        dinv = dinv_ref[...]
        s1 = dinv * (tp_ref[0] + tp_ref[1] + g_ref[...])
        acc_a = jnp.zeros_like(s1)
        acc_b = jnp.zeros_like(s1)
        for k in range(16):
            h = jnp.maximum(s1 * w1_ref[0, k] + b1_ref[0, k], 0.0)
            acc_a = acc_a + h * w2_ref[k, 0]
            acc_b = acc_b + h * w2_ref[k, 1]
        a_ref[...] = dinv * acc_a
        b_ref[...] = dinv * acc_b

    r = g.shape[0]
    smem = pl.BlockSpec(memory_space=pltpu.MemorySpace.SMEM)
    vmem = pl.BlockSpec(memory_space=pltpu.MemorySpace.VMEM)
    return pl.pallas_call(
        body,
        in_specs=[vmem, vmem, vmem, smem, smem, smem],
        out_shape=(
            jax.ShapeDtypeStruct((r, 128), jnp.float32),
            jax.ShapeDtypeStruct((r, 128), jnp.float32),
        ),
    )(tp, g, dinv, W1, b1, W2)


def _tc_final(t2ap, t2bp, g2a, g2b, dinv, b2):
    def body(ta_ref, tb_ref, ga_ref, gb_ref, dinv_ref, b2_ref, oa_ref, ob_ref):
        dinv = dinv_ref[...]
        a = dinv * (ta_ref[0] + ta_ref[1] + ga_ref[...]) + b2_ref[0, 0]
        b = dinv * (tb_ref[0] + tb_ref[1] + gb_ref[...]) + b2_ref[0, 1]
        m = jnp.maximum(a, b)
        lse = m + jnp.log(jnp.exp(a - m) + jnp.exp(b - m))
        oa_ref[...] = a - lse
        ob_ref[...] = b - lse

    r = g2a.shape[0]
    smem = pl.BlockSpec(memory_space=pltpu.MemorySpace.SMEM)
    vmem = pl.BlockSpec(memory_space=pltpu.MemorySpace.VMEM)
    return pl.pallas_call(
        body,
        in_specs=[vmem, vmem, vmem, vmem, vmem, smem],
        out_shape=(
            jax.ShapeDtypeStruct((r, 128), jnp.float32),
            jax.ShapeDtypeStruct((r, 128), jnp.float32),
        ),
    )(t2ap, t2bp, g2a, g2b, dinv, b2)


def kernel(x, edge_index, W1, b1, W2, b2):
    n = x.shape[0]
    e = edge_index.shape[1]
    n_pad = ((n + 1023) // 1024) * 1024
    r = n_pad // 128
    nw = NC * NS
    c_sz = 2048
    nb = 7
    epw = ((e + nw - 1) // nw + c_sz * nb - 1) // (c_sz * nb) * (c_sz * nb)
    e_pad = nw * epw

    src = edge_index[0].astype(jnp.int32)
    dst = edge_index[1].astype(jnp.int32)
    pad = e_pad - e
    pad_idx = (n + jnp.arange(pad, dtype=jnp.int32) % (n_pad - n)).astype(jnp.int32)
    src_p = jnp.concatenate([src, pad_idx])
    dst_p = jnp.concatenate([dst, pad_idx])

    zeros_np = jnp.zeros((n_pad,), jnp.float32)
    xp = jnp.pad(x[:, 0], (0, n_pad - n)).reshape(r, 128)

    ones_c = jnp.ones((c_sz,), jnp.float32)
    degp = _make_deg_kernel(n_pad, epw, c_sz, nb)(dst_p, ones_c, zeros_np)

    dinv, g = _tc_norm(degp.reshape(NC, r, 128), xp)

    tp = _make_edge_kernel(n_pad, epw, c_sz, 1, nb)(
        src_p, dst_p, g.reshape(n_pad), zeros_np
    )

    g2a, g2b = _tc_layer1(
        tp.reshape(NC, r, 128), g, dinv,
        W1.reshape(1, 16), b1.reshape(1, 16), W2,
    )

    t2ap, t2bp = _make_edge_kernel(n_pad, epw, c_sz, 2, nb)(
        src_p, dst_p, g2a.reshape(n_pad), g2b.reshape(n_pad), zeros_np
    )

    oa, ob = _tc_final(
        t2ap.reshape(NC, r, 128), t2bp.reshape(NC, r, 128),
        g2a, g2b, dinv, b2.reshape(1, 2),
    )
    return jnp.stack([oa.reshape(n_pad)[:n], ob.reshape(n_pad)[:n]], axis=1)
